# Initial kernel scaffold; baseline (speedup 1.0000x reference)
import jax
import jax.numpy as jnp
from jax.experimental import pallas as pl


def kernel(num_group, table):
    raise NotImplementedError("write your pallas kernel here")


# speedup vs baseline: 6.2011x; 6.2011x over previous
import functools

import jax
import jax.numpy as jnp
from jax import lax
from jax.experimental import pallas as pl
from jax.experimental.pallas import tpu as pltpu
from jax.experimental.pallas import tpu_sc as plsc

NUM_GROUP = 100000
EMBED_DIM = 64
BATCH = 16384
HIST = 50

_INFO = plsc.get_sparse_core_info()
NC = _INFO.num_cores
NS = _INFO.num_subcores
NW = NC * NS

B = BATCH * HIST
B_PER_W = B // NW
CHUNK = 128
N_CHUNK = B_PER_W // CHUNK
NBUF = 4
N_GROUP = N_CHUNK // NBUF


def _body(idx_hbm, table_hbm, out_hbm, idx_v, rows_v, gsem, ssem):
    c = lax.axis_index("c")
    s = lax.axis_index("s")
    wid = s * NC + c
    base = wid * B_PER_W

    pltpu.sync_copy(idx_hbm.at[wid], idx_v)

    def group(g, carry):
        c0 = g * NBUF
        gathers = []
        for b in range(NBUF):
            gathers.append(
                pltpu.async_copy(
                    table_hbm.at[idx_v.at[c0 + b]], rows_v.at[b], gsem
                )
            )
        stores = []
        for b in range(NBUF):
            gathers[b].wait()
            stores.append(
                pltpu.async_copy(
                    rows_v.at[b],
                    out_hbm.at[pl.ds(base + (c0 + b) * CHUNK, CHUNK)],
                    ssem,
                )
            )
        for b in range(NBUF):
            stores[b].wait()
        return carry

    lax.fori_loop(0, N_GROUP, group, 0)


@jax.jit
def _lookup(idx, table):
    kern = pl.kernel(
        _body,
        out_type=jax.ShapeDtypeStruct((B, EMBED_DIM), jnp.float32),
        mesh=plsc.VectorSubcoreMesh(core_axis_name="c", subcore_axis_name="s"),
        scratch_types=[
            pltpu.VMEM((N_CHUNK, CHUNK), jnp.int32),
            pltpu.VMEM((NBUF, CHUNK, EMBED_DIM), jnp.float32),
            pltpu.SemaphoreType.DMA,
            pltpu.SemaphoreType.DMA,
        ],
        compiler_params=pltpu.CompilerParams(use_tc_tiling_on_sc=False),
    )
    return kern(idx, table)


def kernel(num_group, table):
    idx = num_group.astype(jnp.int32).reshape(NW, N_CHUNK, CHUNK)
    out = _lookup(idx, table)
    return out.reshape(BATCH, HIST, EMBED_DIM)


# speedup vs baseline: 6.2191x; 1.0029x over previous
import functools

import jax
import jax.numpy as jnp
from jax import lax
from jax.experimental import pallas as pl
from jax.experimental.pallas import tpu as pltpu
from jax.experimental.pallas import tpu_sc as plsc

NUM_GROUP = 100000
EMBED_DIM = 64
BATCH = 16384
HIST = 50

_INFO = plsc.get_sparse_core_info()
NC = _INFO.num_cores
NS = _INFO.num_subcores
NW = NC * NS

B = BATCH * HIST
B_PER_W = B // NW
CHUNK = 128
N_CHUNK = B_PER_W // CHUNK
NBUF = 8
N_GROUP = N_CHUNK // NBUF


def _body(idx_hbm, table_hbm, out_hbm, idx_v, rows_v, gsem, ssem):
    c = lax.axis_index("c")
    s = lax.axis_index("s")
    wid = s * NC + c
    base = wid * B_PER_W

    pltpu.sync_copy(idx_hbm.at[wid], idx_v)

    def fire_gathers(c0):
        for b in range(NBUF):
            pltpu.async_copy(table_hbm.at[idx_v.at[c0 + b]], rows_v.at[b], gsem)

    def wait_and_store(c0):
        for b in range(NBUF):
            pltpu.make_async_copy(
                table_hbm.at[idx_v.at[c0 + b]], rows_v.at[b], gsem
            ).wait()
            pltpu.async_copy(
                rows_v.at[b],
                out_hbm.at[pl.ds(base + (c0 + b) * CHUNK, CHUNK)],
                ssem,
            )

    def drain_stores():
        for b in range(NBUF):
            pltpu.make_async_copy(
                rows_v.at[b], out_hbm.at[pl.ds(base, CHUNK)], ssem
            ).wait()

    fire_gathers(0)
    wait_and_store(0)

    def group(g, carry):
        c0 = g * NBUF
        drain_stores()
        fire_gathers(c0)
        wait_and_store(c0)
        return carry

    lax.fori_loop(1, N_GROUP, group, 0)
    drain_stores()


@jax.jit
def _lookup(idx, table):
    kern = pl.kernel(
        _body,
        out_type=jax.ShapeDtypeStruct((B, EMBED_DIM), jnp.float32),
        mesh=plsc.VectorSubcoreMesh(core_axis_name="c", subcore_axis_name="s"),
        scratch_types=[
            pltpu.VMEM((N_CHUNK, CHUNK), jnp.int32),
            pltpu.VMEM((NBUF, CHUNK, EMBED_DIM), jnp.float32),
            pltpu.SemaphoreType.DMA,
            pltpu.SemaphoreType.DMA,
        ],
        compiler_params=pltpu.CompilerParams(use_tc_tiling_on_sc=False),
    )
    return kern(idx, table)


def kernel(num_group, table):
    idx = num_group.astype(jnp.int32).reshape(NW, N_CHUNK, CHUNK)
    out = _lookup(idx, table)
    return out.reshape(BATCH, HIST, EMBED_DIM)
